# baseline (speedup 1.0000x reference)
import functools

import jax
import jax.numpy as jnp
from jax import lax
from jax.experimental import pallas as pl
from jax.experimental.pallas import tpu as pltpu
from jax.experimental.pallas import tpu_sc as plsc

NC = 2
NS = 16
NW = NC * NS
L = 16

NQ = 256
TBL = 272
TSTRIDE = 273
TREP = 16

ROWS = 4096
COLS = 8192
ROWS_W = ROWS // NW
BR = 8
BC = 2048
BLK = BR * BC
CBLK = COLS // BC
NBLK = (ROWS_W // BR) * CBLK


def _sc_body(x_hbm, q_hbm, bp_hbm, out_hbm,
             ibuf0, ibuf1, obuf0, obuf1,
             qv, bpv, tbl_a, tbl_b,
             sin0, sin1, sout0, sout1):
  wid = lax.axis_index("s") * NC + lax.axis_index("c")
  row0 = wid * ROWS_W

  def hbm_slice(ref, g):
    r = row0 + (g // CBLK) * BR
    c = (g % CBLK) * BC
    return ref.at[pl.ds(r, BR), pl.ds(c, BC)]

  def in_copy(g, buf, sem):
    return pltpu.make_async_copy(hbm_slice(x_hbm, g), buf, sem)

  def out_copy(g, buf, sem):
    return pltpu.make_async_copy(buf, hbm_slice(out_hbm, g), sem)

  in_copy(0, ibuf0, sin0).start()
  in_copy(1, ibuf1, sin1).start()

  pltpu.sync_copy(q_hbm, qv)
  pltpu.sync_copy(bp_hbm, bpv)

  iota = lax.iota(jnp.int32, L)

  @pl.loop(0, TBL // L)
  def _build(i):
    kv = i * L + iota
    kc = jnp.minimum(kv, NQ)
    km1 = jnp.where(kc == 0, NQ - 1, kc - 1)
    kcl = jnp.minimum(kc, NQ - 1)
    q_k = plsc.load_gather(qv, [kcl])
    q_m = plsc.load_gather(qv, [km1])
    b_k = plsc.load_gather(bpv, [kcl])
    b_m = plsc.load_gather(bpv, [km1])
    interior = (kc >= 1) & (kc <= NQ - 1)
    denom = jnp.where(interior, b_k - b_m, 1.0)
    a = jnp.where(interior, (q_k - q_m) / denom, 0.0)
    b = q_m - b_m * a
    for c in range(TREP):
      plsc.store_scatter(tbl_a, [kv + c * TSTRIDE], a)
      plsc.store_scatter(tbl_b, [kv + c * TSTRIDE], b)

  def compute(ibuf, obuf):
    vregs_per_row = BC // L
    lane_off = iota * TSTRIDE
    lane_off1 = lane_off + 1

    @plsc.parallel_loop(0, BLK // L, 1, unroll=16)
    def _vec(i):
      r = i // vregs_per_row
      off = (i % vregs_per_row) * L
      xv = ibuf[r, pl.ds(off, L)]
      u = (xv + 1.0) * 127.5
      uc = jnp.minimum(jnp.maximum(u, 0.0), 256.0)
      t = uc.astype(jnp.int32)
      k = t + jnp.where(t.astype(jnp.float32) < uc, lane_off1, lane_off)
      a = plsc.load_gather(tbl_a, [k])
      b = plsc.load_gather(tbl_b, [k])
      obuf[r, pl.ds(off, L)] = xv * a + b

  ibufs = (ibuf0, ibuf1)
  obufs = (obuf0, obuf1)
  sins = (sin0, sin1)
  souts = (sout0, sout1)

  @pl.loop(0, NBLK // 2)
  def _outer(g2):
    for p in range(2):
      g = g2 * 2 + p
      in_copy(g, ibufs[p], sins[p]).wait()

      @pl.when(g >= 2)
      def _():
        out_copy(g - 2, obufs[p], souts[p]).wait()

      compute(ibufs[p], obufs[p])
      out_copy(g, obufs[p], souts[p]).start()

      @pl.when(g + 2 < NBLK)
      def _():
        in_copy(g + 2, ibufs[p], sins[p]).start()

  out_copy(NBLK - 2, obuf0, sout0).wait()
  out_copy(NBLK - 1, obuf1, sout1).wait()


@jax.jit
def _sc_call(x, quantiles, breakpoints):
  mesh = plsc.VectorSubcoreMesh(core_axis_name="c", subcore_axis_name="s",
                                num_cores=NC, num_subcores=NS)
  f = pl.kernel(
      _sc_body,
      out_type=jax.ShapeDtypeStruct((ROWS, COLS), jnp.float32),
      mesh=mesh,
      compiler_params=pltpu.CompilerParams(needs_layout_passes=False),
      scratch_types=[
          pltpu.VMEM((BR, BC), jnp.float32),
          pltpu.VMEM((BR, BC), jnp.float32),
          pltpu.VMEM((BR, BC), jnp.float32),
          pltpu.VMEM((BR, BC), jnp.float32),
          pltpu.VMEM((NQ,), jnp.float32),
          pltpu.VMEM((NQ,), jnp.float32),
          pltpu.VMEM((TSTRIDE * TREP,), jnp.float32),
          pltpu.VMEM((TSTRIDE * TREP,), jnp.float32),
          pltpu.SemaphoreType.DMA,
          pltpu.SemaphoreType.DMA,
          pltpu.SemaphoreType.DMA,
          pltpu.SemaphoreType.DMA,
      ],
  )
  return f(x, quantiles, breakpoints)


def kernel(x, quantiles, breakpoints):
  return _sc_call(x, quantiles, breakpoints)


# speedup vs baseline: 1.1526x; 1.1526x over previous
import functools

import jax
import jax.numpy as jnp
from jax import lax
from jax.experimental import pallas as pl
from jax.experimental.pallas import tpu as pltpu
from jax.experimental.pallas import tpu_sc as plsc

NC = 2
NS = 16
NW = NC * NS
L = 16

NQ = 256
TBL = 272
TSTRIDE = 273
TREP = 16

ROWS = 4096
COLS = 8192
ROWS_W = ROWS // NW
BR = 8
BC = 2048
BLK = BR * BC
CBLK = COLS // BC
NBLK = (ROWS_W // BR) * CBLK


def _sc_body(x_hbm, q_hbm, bp_hbm, out_hbm,
             ibuf0, ibuf1, obuf0, obuf1,
             qv, bpv, tbl_a, tbl_b,
             sin0, sin1, sout0, sout1):
  wid = lax.axis_index("s") * NC + lax.axis_index("c")
  row0 = wid * ROWS_W

  def hbm_slice(ref, g):
    r = row0 + (g // CBLK) * BR
    c = (g % CBLK) * BC
    return ref.at[pl.ds(r, BR), pl.ds(c, BC)]

  def in_copy(g, buf, sem):
    return pltpu.make_async_copy(hbm_slice(x_hbm, g), buf, sem)

  def out_copy(g, buf, sem):
    return pltpu.make_async_copy(buf, hbm_slice(out_hbm, g), sem)

  in_copy(0, ibuf0, sin0).start()
  in_copy(1, ibuf1, sin1).start()

  pltpu.sync_copy(q_hbm, qv)
  pltpu.sync_copy(bp_hbm, bpv)

  iota = lax.iota(jnp.int32, L)

  @pl.loop(0, TBL // L)
  def _build(i):
    kv = i * L + iota
    kc = jnp.minimum(kv, NQ)
    km1 = jnp.where(kc == 0, NQ - 1, kc - 1)
    kcl = jnp.minimum(kc, NQ - 1)
    q_k = plsc.load_gather(qv, [kcl])
    q_m = plsc.load_gather(qv, [km1])
    b_k = plsc.load_gather(bpv, [kcl])
    b_m = plsc.load_gather(bpv, [km1])
    interior = (kc >= 1) & (kc <= NQ - 1)
    denom = jnp.where(interior, b_k - b_m, 1.0)
    a = jnp.where(interior, (q_k - q_m) / denom, 0.0)
    b = q_m - b_m * a
    for c in range(TREP):
      plsc.store_scatter(tbl_a, [kv + c * TSTRIDE], a)
      plsc.store_scatter(tbl_b, [kv + c * TSTRIDE], b)

  def compute(ibuf, obuf):
    vregs_per_row = BC // L
    lane_off = iota * TSTRIDE
    lane_off1 = lane_off + 1

    @plsc.parallel_loop(0, BLK // L, 1, unroll=8)
    def _vec(i):
      r = i // vregs_per_row
      off = (i % vregs_per_row) * L
      xv = ibuf[r, pl.ds(off, L)]
      u = (xv + 1.0) * 127.5
      uc = jnp.minimum(jnp.maximum(u, 0.0), 256.0)
      t = uc.astype(jnp.int32)
      k = t + jnp.where(t.astype(jnp.float32) < uc, lane_off1, lane_off)
      a = plsc.load_gather(tbl_a, [k])
      b = plsc.load_gather(tbl_b, [k])
      obuf[r, pl.ds(off, L)] = xv * a + b

  ibufs = (ibuf0, ibuf1)
  obufs = (obuf0, obuf1)
  sins = (sin0, sin1)
  souts = (sout0, sout1)

  @pl.loop(0, NBLK // 2)
  def _outer(g2):
    for p in range(2):
      g = g2 * 2 + p
      in_copy(g, ibufs[p], sins[p]).wait()

      @pl.when(g >= 2)
      def _():
        out_copy(g - 2, obufs[p], souts[p]).wait()

      compute(ibufs[p], obufs[p])
      out_copy(g, obufs[p], souts[p]).start()

      @pl.when(g + 2 < NBLK)
      def _():
        in_copy(g + 2, ibufs[p], sins[p]).start()

  out_copy(NBLK - 2, obuf0, sout0).wait()
  out_copy(NBLK - 1, obuf1, sout1).wait()


@jax.jit
def _sc_call(x, quantiles, breakpoints):
  mesh = plsc.VectorSubcoreMesh(core_axis_name="c", subcore_axis_name="s",
                                num_cores=NC, num_subcores=NS)
  f = pl.kernel(
      _sc_body,
      out_type=jax.ShapeDtypeStruct((ROWS, COLS), jnp.float32),
      mesh=mesh,
      compiler_params=pltpu.CompilerParams(needs_layout_passes=False),
      scratch_types=[
          pltpu.VMEM((BR, BC), jnp.float32),
          pltpu.VMEM((BR, BC), jnp.float32),
          pltpu.VMEM((BR, BC), jnp.float32),
          pltpu.VMEM((BR, BC), jnp.float32),
          pltpu.VMEM((NQ,), jnp.float32),
          pltpu.VMEM((NQ,), jnp.float32),
          pltpu.VMEM((TSTRIDE * TREP,), jnp.float32),
          pltpu.VMEM((TSTRIDE * TREP,), jnp.float32),
          pltpu.SemaphoreType.DMA,
          pltpu.SemaphoreType.DMA,
          pltpu.SemaphoreType.DMA,
          pltpu.SemaphoreType.DMA,
      ],
  )
  return f(x, quantiles, breakpoints)


def kernel(x, quantiles, breakpoints):
  return _sc_call(x, quantiles, breakpoints)
